# Initial kernel scaffold; baseline (speedup 1.0000x reference)
import jax
import jax.numpy as jnp
from jax.experimental import pallas as pl


def kernel(x, edge_list, W1_l, b1, W1_r, W2_l, b2, W2_r, W_lin1, b_lin1, W_lin2, b_lin2):
    raise NotImplementedError("write your pallas kernel here")


# speedup vs baseline: 7.2319x; 7.2319x over previous
import functools

import jax
import jax.numpy as jnp
from jax import lax
from jax.experimental import pallas as pl
from jax.experimental.pallas import tpu as pltpu
from jax.experimental.pallas import tpu_sc as plsc

N_NODES = 10000
D_FEAT = 128
HIDDEN = 64
NUM_CLASSES = 2

N_PAD = 10240
NW = 32
BATCH = 128
DEG_W = 16
ROW_BLK = 1000


def _cdiv(a, b):
    return (a + b - 1) // b


def _zero_rows(buf, n_rows, width):
    per_row = width // 16

    def z(i, carry):
        r = i // per_row
        c = (i % per_row) * 16
        buf[r, pl.ds(c, 16)] = jnp.zeros((16,), jnp.float32)
        return carry

    lax.fori_loop(0, n_rows * per_row, z, 0)


def _make_edge_pass(k, with_deg):
    mesh = plsc.VectorSubcoreMesh(core_axis_name="c", subcore_axis_name="s")
    rows_per_tile = N_PAD // 16

    out_type = [jax.ShapeDtypeStruct((2, N_PAD, HIDDEN), jnp.float32)]
    scratch = [
        pltpu.VMEM((k, BATCH), jnp.int32),
        pltpu.VMEM((k, BATCH), jnp.int32),
        pltpu.VMEM((BATCH, HIDDEN), jnp.float32),
        pltpu.VMEM((16, HIDDEN), jnp.float32),
        pltpu.VMEM_SHARED((N_PAD, HIDDEN), jnp.float32),
        pltpu.SemaphoreType.DMA,
    ]
    if with_deg:
        out_type.append(jax.ShapeDtypeStruct((2, N_PAD, DEG_W), jnp.float32))
        scratch += [
            pltpu.VMEM((BATCH, DEG_W), jnp.float32),
            pltpu.VMEM((16, DEG_W), jnp.float32),
            pltpu.VMEM_SHARED((N_PAD, DEG_W), jnp.float32),
        ]

    def body(table, src3, dst3, *refs):
        if with_deg:
            (out_acc, out_deg, srcv, dstv, rows, zacc, acc_sh, sem,
             ones_v, zdeg, deg_sh) = refs
        else:
            out_acc, srcv, dstv, rows, zacc, acc_sh, sem = refs
        c = lax.axis_index("c")
        s = lax.axis_index("s")
        wid = s * 2 + c
        base = s * rows_per_tile

        _zero_rows(zacc, 16, HIDDEN)
        if with_deg:
            _zero_rows(zdeg, 16, DEG_W)

            def ofill(i, carry):
                ones_v[i, pl.ds(0, 16)] = jnp.full((16,), 1.0, jnp.float32)
                return carry

            lax.fori_loop(0, BATCH, ofill, 0)

        def zc(i, carry):
            pltpu.sync_copy(zacc, acc_sh.at[pl.ds(base + i * 16, 16)])
            if with_deg:
                pltpu.sync_copy(zdeg, deg_sh.at[pl.ds(base + i * 16, 16)])
            return carry

        lax.fori_loop(0, rows_per_tile // 16, zc, 0)

        pltpu.sync_copy(src3.at[wid], srcv)
        pltpu.sync_copy(dst3.at[wid], dstv)
        plsc.subcore_barrier()

        def step(j, carry):
            pltpu.async_copy(table.at[srcv.at[j]], rows, sem).wait()
            pltpu.sync_copy(rows, acc_sh.at[dstv.at[j]], add=True)
            if with_deg:
                pltpu.sync_copy(ones_v, deg_sh.at[dstv.at[j]], add=True)
            return carry

        lax.fori_loop(0, k, step, 0)
        plsc.subcore_barrier()

        def out_i(i, carry):
            r0 = base + i * BATCH
            pltpu.sync_copy(acc_sh.at[pl.ds(r0, BATCH)], rows)
            pltpu.sync_copy(rows, out_acc.at[c, pl.ds(r0, BATCH)])
            if with_deg:
                pltpu.sync_copy(deg_sh.at[pl.ds(r0, BATCH)], ones_v)
                pltpu.sync_copy(ones_v, out_deg.at[c, pl.ds(r0, BATCH)])
            return carry

        lax.fori_loop(0, rows_per_tile // BATCH, out_i, 0)

    return pl.kernel(body, out_type=out_type, mesh=mesh,
                     scratch_types=scratch,
                     compiler_params=pltpu.CompilerParams(
                         use_tc_tiling_on_sc=False))


def _proj_body(x_ref, wl_ref, wr_ref, p_ref, r_ref):
    xv = x_ref[...]
    p_ref[...] = jnp.dot(xv, wl_ref[...], preferred_element_type=jnp.float32)
    r_ref[...] = jnp.dot(xv, wr_ref[...], preferred_element_type=jnp.float32)


def _mid_body(acc_ref, deg_ref, r1_ref, b1_ref, wl_ref, wr_ref, p_ref, r_ref):
    acc = acc_ref[0] + acc_ref[1]
    deg = deg_ref[0] + deg_ref[1]
    mean = acc / jnp.maximum(deg[:, :1], 1.0)
    h = jax.nn.sigmoid(mean + b1_ref[...] + r1_ref[...])
    p_ref[...] = jnp.dot(h, wl_ref[...], preferred_element_type=jnp.float32)
    r_ref[...] = jnp.dot(h, wr_ref[...], preferred_element_type=jnp.float32)


def _fin_body(acc_ref, deg_ref, r2_ref, b2_ref, wl1_ref, bl1_ref, wl2_ref,
              bl2_ref, out_ref):
    acc = acc_ref[0] + acc_ref[1]
    deg = deg_ref[0] + deg_ref[1]
    mean = acc / jnp.maximum(deg[:, :1], 1.0)
    h = jax.nn.sigmoid(mean + b2_ref[...] + r2_ref[...])
    t = jax.nn.sigmoid(
        jnp.dot(h, wl1_ref[...], preferred_element_type=jnp.float32)
        + bl1_ref[...])
    logits = (jnp.dot(t, wl2_ref[...], preferred_element_type=jnp.float32)
              + bl2_ref[...])
    m = jnp.max(logits, axis=1, keepdims=True)
    e = jnp.exp(logits - m)
    out_ref[...] = e / jnp.sum(e, axis=1, keepdims=True)


def _blk(shape, idx):
    return pl.BlockSpec(shape, idx)


_GRID = N_NODES // ROW_BLK


def _proj(x, wl, wr, d_in):
    return pl.pallas_call(
        _proj_body,
        grid=(_GRID,),
        in_specs=[
            _blk((ROW_BLK, d_in), lambda i: (i, 0)),
            _blk((d_in, HIDDEN), lambda i: (0, 0)),
            _blk((d_in, HIDDEN), lambda i: (0, 0)),
        ],
        out_specs=[
            _blk((ROW_BLK, HIDDEN), lambda i: (i, 0)),
            _blk((ROW_BLK, HIDDEN), lambda i: (i, 0)),
        ],
        out_shape=[jax.ShapeDtypeStruct((N_NODES, HIDDEN), jnp.float32)] * 2,
    )(x, wl, wr)


def _mid(acc, deg, r1, b1, wl, wr):
    return pl.pallas_call(
        _mid_body,
        grid=(_GRID,),
        in_specs=[
            _blk((2, ROW_BLK, HIDDEN), lambda i: (0, i, 0)),
            _blk((2, ROW_BLK, DEG_W), lambda i: (0, i, 0)),
            _blk((ROW_BLK, HIDDEN), lambda i: (i, 0)),
            _blk((1, HIDDEN), lambda i: (0, 0)),
            _blk((HIDDEN, HIDDEN), lambda i: (0, 0)),
            _blk((HIDDEN, HIDDEN), lambda i: (0, 0)),
        ],
        out_specs=[
            _blk((ROW_BLK, HIDDEN), lambda i: (i, 0)),
            _blk((ROW_BLK, HIDDEN), lambda i: (i, 0)),
        ],
        out_shape=[jax.ShapeDtypeStruct((N_NODES, HIDDEN), jnp.float32)] * 2,
    )(acc, deg, r1, b1, wl, wr)


def _fin(acc, deg, r2, b2, wl1, bl1, wl2, bl2):
    return pl.pallas_call(
        _fin_body,
        grid=(_GRID,),
        in_specs=[
            _blk((2, ROW_BLK, HIDDEN), lambda i: (0, i, 0)),
            _blk((2, ROW_BLK, DEG_W), lambda i: (0, i, 0)),
            _blk((ROW_BLK, HIDDEN), lambda i: (i, 0)),
            _blk((1, HIDDEN), lambda i: (0, 0)),
            _blk((HIDDEN, HIDDEN), lambda i: (0, 0)),
            _blk((1, HIDDEN), lambda i: (0, 0)),
            _blk((HIDDEN, NUM_CLASSES), lambda i: (0, 0)),
            _blk((1, NUM_CLASSES), lambda i: (0, 0)),
        ],
        out_specs=_blk((ROW_BLK, NUM_CLASSES), lambda i: (i, 0)),
        out_shape=jax.ShapeDtypeStruct((N_NODES, NUM_CLASSES), jnp.float32),
    )(acc, deg, r2, b2, wl1, bl1, wl2, bl2)


def kernel(x, edge_list, W1_l, b1, W1_r, W2_l, b2, W2_r,
           W_lin1, b_lin1, W_lin2, b_lin2):
    n_edges = edge_list.shape[1]
    k = _cdiv(n_edges, NW * BATCH)
    e_pad = NW * k * BATCH - n_edges

    src = edge_list[0].astype(jnp.int32)
    dst = edge_list[1].astype(jnp.int32)
    src3 = jnp.pad(src, (0, e_pad)).reshape(NW, k, BATCH)
    dst3 = jnp.pad(dst, (0, e_pad),
                   constant_values=N_NODES).reshape(NW, k, BATCH)

    edge_pass_deg = _make_edge_pass(k, with_deg=True)
    edge_pass = _make_edge_pass(k, with_deg=False)

    b1_2 = b1.reshape(1, HIDDEN)
    b2_2 = b2.reshape(1, HIDDEN)
    bl1_2 = b_lin1.reshape(1, HIDDEN)
    bl2_2 = b_lin2.reshape(1, NUM_CLASSES)

    p1, r1 = _proj(x, W1_l, W1_r, D_FEAT)
    acc1, deg = edge_pass_deg(p1, src3, dst3)
    p2, r2 = _mid(acc1, deg, r1, b1_2, W2_l, W2_r)

    (acc2,) = (edge_pass(p2, src3, dst3),)
    if isinstance(acc2, (list, tuple)):
        acc2 = acc2[0]
    return _fin(acc2, deg, r2, b2_2, W_lin1, bl1_2, W_lin2, bl2_2)
